# baseline (speedup 1.0000x reference)
import functools
import math

import jax
import jax.numpy as jnp
import numpy as np
from jax import lax
from jax.experimental import pallas as pl
from jax.experimental.pallas import tpu as pltpu
from jax.experimental.pallas import tpu_sc as plsc

ALPHA = 0.1
THETA = 0.5
CHUNK = 128
NBUF = 2
NSRC = 4


def _sc_info():
    try:
        info = plsc.get_sparse_core_info()
        return info.num_cores, info.num_subcores
    except Exception:
        return 2, 16


@functools.lru_cache(maxsize=None)
def _make_segment_sum(N, D, chunks_per_tile):
    NC, NS = _sc_info()
    n_full = N // CHUNK
    rem = N - n_full * CHUNK
    row_iters = math.ceil((n_full + (1 if rem else 0)) / NS)
    N_acc = N + 8
    outer = chunks_per_tile // NSRC
    mesh = plsc.VectorSubcoreMesh(core_axis_name="c", subcore_axis_name="s")

    @functools.partial(
        pl.kernel,
        mesh=mesh,
        out_type=jax.ShapeDtypeStruct((NC, N, D), jnp.float32),
        scratch_types=[
            pltpu.VMEM((chunks_per_tile, CHUNK), jnp.int32),
            pltpu.VMEM((NSRC * CHUNK,), jnp.int32),
        ]
        + [pltpu.VMEM((CHUNK, D), jnp.float32) for _ in range(NBUF)]
        + [pltpu.VMEM_SHARED((N_acc, D), jnp.float32)]
        + [pltpu.SemaphoreType.DMA for _ in range(NBUF)]
        + [pltpu.SemaphoreType.DMA for _ in range(NSRC)]
        + [pltpu.SemaphoreType.DMA],
    )
    def seg(h_hbm, src_hbm, dst_hbm, zeros_hbm, out_hbm,
            dst_v, src_v, *rest):
        rows = rest[:NBUF]
        acc = rest[NBUF]
        gsem = rest[NBUF + 1:2 * NBUF + 1]
        ssem = rest[2 * NBUF + 1:2 * NBUF + 1 + NSRC]
        isem = rest[2 * NBUF + 1 + NSRC]
        c = lax.axis_index("c")
        s = lax.axis_index("s")
        w = s * NC + c
        edge_base = w * (chunks_per_tile * CHUNK)

        def for_each_row_block(fn, include_dummy=False):
            for i in range(row_iters):
                b = s + NS * i

                @pl.when(b < n_full)
                def _():
                    fn(b * CHUNK, CHUNK)

                if rem:
                    sz = rem + (8 if include_dummy else 0)

                    @pl.when(b == n_full)
                    def _():
                        fn(n_full * CHUNK, sz)

        def src_slot(slot):
            return src_v.at[pl.ds(slot * CHUNK, CHUNK)]

        def prefetch_src(i, slot):
            pltpu.async_copy(
                src_hbm.at[pl.ds(edge_base + i * CHUNK, CHUNK)],
                src_slot(slot), ssem[slot])

        def wait_src(i, slot):
            pltpu.make_async_copy(
                src_hbm.at[pl.ds(edge_base + i * CHUNK, CHUNK)],
                src_slot(slot), ssem[slot]).wait()

        def gather(i, slot, b):
            pltpu.async_copy(h_hbm.at[src_slot(slot)], rows[b], gsem[b])

        def wait_gather(slot, b):
            pltpu.make_async_copy(h_hbm.at[src_slot(slot)], rows[b],
                                  gsem[b]).wait()

        pltpu.async_copy(dst_hbm.at[w], dst_v, isem)
        for j in range(NSRC):
            prefetch_src(j, j)
        for_each_row_block(lambda base, sz: pltpu.sync_copy(
            zeros_hbm.at[pl.ds(0, sz)], acc.at[pl.ds(base, sz)]),
            include_dummy=True)
        pltpu.make_async_copy(dst_hbm.at[w], dst_v, isem).wait()
        plsc.subcore_barrier()

        for b in range(NBUF):
            wait_src(b, b)
            gather(b, b, b)

        def body(g, carry):
            for b in range(NSRC):
                i = g * NSRC + b
                rb = b % NBUF
                wait_gather(b, rb)
                pltpu.sync_copy(rows[rb], acc.at[dst_v.at[i]], add=True)

                @pl.when(g + 1 < outer)
                def _():
                    prefetch_src(i + NSRC, b)

                nb = (b + NBUF) % NSRC

                @pl.when(i + NBUF < chunks_per_tile)
                def _():
                    wait_src(i + NBUF, nb)
                    gather(i + NBUF, nb, rb)
            return carry

        lax.fori_loop(0, outer, body, None)
        plsc.subcore_barrier()
        for_each_row_block(lambda base, sz: pltpu.sync_copy(
            acc.at[pl.ds(base, sz)], out_hbm.at[c, pl.ds(base, sz)]))

    return seg


def _mm_relu_body(x_ref, w_ref, b_ref, o_ref):
    y = jnp.dot(x_ref[...], w_ref[...], preferred_element_type=jnp.float32)
    o_ref[...] = jnp.maximum(y + b_ref[...], 0.0)


def _combine_body(p0_ref, p1_ref, x0_ref, w_ref, o_ref, *, beta):
    z = (1.0 - ALPHA) * (p0_ref[...] + p1_ref[...]) + ALPHA * x0_ref[...]
    y = (1.0 - beta) * z + beta * jnp.dot(z, w_ref[...], preferred_element_type=jnp.float32)
    o_ref[...] = jnp.maximum(y, 0.0)


def _final_body(h_ref, w_ref, b_ref, o_ref, *, C):
    logits = jnp.dot(h_ref[...], w_ref[...], preferred_element_type=jnp.float32) + b_ref[...]
    col = lax.broadcasted_iota(jnp.int32, logits.shape, 1)
    valid = col < C
    masked = jnp.where(valid, logits, -jnp.inf)
    m = jnp.max(masked, axis=1, keepdims=True)
    ex = jnp.where(valid, jnp.exp(masked - m), 0.0)
    lse = jnp.log(jnp.sum(ex, axis=1, keepdims=True)) + m
    o_ref[...] = logits - lse


def _tc_call(body, out_shape, *args):
    return pl.pallas_call(body, out_shape=out_shape)(*args)


def kernel(x, edge_index, W0, b0, Wc, W1, b1):
    N, D = x.shape
    H = W0.shape[1]
    C = W1.shape[1]
    L = Wc.shape[0]
    E = edge_index.shape[1]
    NC, NS = _sc_info()
    NW = NC * NS

    per_tile = math.ceil(E / (NW * CHUNK * NSRC)) * CHUNK * NSRC
    E_pad = per_tile * NW
    chunks_per_tile = per_tile // CHUNK
    src = edge_index[0].astype(jnp.int32)
    dst = edge_index[1].astype(jnp.int32)
    pad = E_pad - E
    src_p = jnp.concatenate([src, jnp.zeros((pad,), jnp.int32)])
    dst_p = jnp.concatenate([dst, jnp.full((pad,), N, jnp.int32)])
    dst3 = dst_p.reshape(NW, chunks_per_tile, CHUNK)
    zeros = jnp.zeros((CHUNK, H), jnp.float32)

    f32 = jnp.float32
    h = _tc_call(_mm_relu_body, jax.ShapeDtypeStruct((N, H), f32),
                 x, W0, b0.reshape(1, H))
    x0 = h
    seg = _make_segment_sum(N, H, chunks_per_tile)
    for l in range(L):
        beta = float(np.log(THETA / (l + 1) + 1.0))
        partials = seg(h, src_p, dst3, zeros)
        h = _tc_call(functools.partial(_combine_body, beta=beta),
                     jax.ShapeDtypeStruct((N, H), f32),
                     partials[0], partials[1], x0, Wc[l])

    Wp = jnp.zeros((H, 128), f32).at[:, :C].set(W1)
    bp = jnp.zeros((1, 128), f32).at[0, :C].set(b1)
    out = _tc_call(functools.partial(_final_body, C=C),
                   jax.ShapeDtypeStruct((N, 128), f32),
                   h, Wp, bp)
    return out[:, :C]
